# baseline (speedup 1.0000x reference)
import jax
import jax.numpy as jnp
from jax.experimental import pallas as pl
from jax.experimental.pallas import tpu as pltpu

_EPS = 1e-4


def _reduce_kernel(y_ref, m_ref, acc_ref):
    w = (m_ref[...] > 0).astype(jnp.float32)
    yb = y_ref[...]
    dn = (((0,), (0,)), ((), ()))
    wh = w.astype(jnp.bfloat16)
    yh = yb.astype(jnp.bfloat16)
    yl = (yb - yh.astype(jnp.float32)).astype(jnp.bfloat16)
    zb = (yb * yb).astype(jnp.bfloat16)
    s = (
        jax.lax.dot_general(wh, yh, dn, preferred_element_type=jnp.float32)
        + jax.lax.dot_general(wh, yl, dn, preferred_element_type=jnp.float32)
    )
    sq = jax.lax.dot_general(wh, zb, dn, preferred_element_type=jnp.float32)
    n = jnp.sum(w)
    acc_ref[0, 0, :] = s[0, :]
    acc_ref[0, 1, :] = sq[0, :]
    acc_ref[0, 2, :] = jnp.full_like(sq[0, :], n)
    acc_ref[0, 3, :] = jnp.zeros_like(sq[0, :])


def _apply_kernel(acc_ref, g_ref, b_ref, y_ref, m_ref, o_ref):
    part = acc_ref[...]
    tot = jnp.sum(part, axis=0)
    s = tot[0, :]
    sq = tot[1, :]
    n = tot[2, :]
    mean = s / n
    var = (sq - s * mean) / (n - 1.0)
    std = jnp.sqrt(var)
    scale = g_ref[0, :] / (std + _EPS)
    shift = b_ref[0, :] - mean * scale
    yb = y_ref[...]
    sel = m_ref[...] > 0
    o_ref[...] = jnp.where(sel, yb * scale + shift, yb)


def kernel(y, mask, gamma, beta):
    B, T, C = y.shape
    rows = B * T
    y2 = y.reshape(rows, C)
    m2 = mask.reshape(rows, 1)

    R1 = 2048
    R = 2048
    g1 = rows // R1
    grid = rows // R

    acc = pl.pallas_call(
        _reduce_kernel,
        grid=(g1,),
        in_specs=[
            pl.BlockSpec((R1, C), lambda i: (i, 0)),
            pl.BlockSpec((R1, 1), lambda i: (i, 0)),
        ],
        out_specs=pl.BlockSpec((1, 4, C), lambda i: (i, 0, 0)),
        out_shape=jax.ShapeDtypeStruct((g1, 4, C), jnp.float32),
        compiler_params=pltpu.CompilerParams(
            dimension_semantics=("parallel",),
            vmem_limit_bytes=120 * 1024 * 1024,
        ),
    )(y2, m2)

    out = pl.pallas_call(
        _apply_kernel,
        grid=(grid,),
        in_specs=[
            pl.BlockSpec((g1, 4, C), lambda i: (0, 0, 0)),
            pl.BlockSpec((1, C), lambda i: (0, 0)),
            pl.BlockSpec((1, C), lambda i: (0, 0)),
            pl.BlockSpec((R, C), lambda i: (i, 0)),
            pl.BlockSpec((R, 1), lambda i: (i, 0)),
        ],
        out_specs=pl.BlockSpec((R, C), lambda i: (i, 0)),
        out_shape=jax.ShapeDtypeStruct((rows, C), jnp.float32),
        compiler_params=pltpu.CompilerParams(
            dimension_semantics=("parallel",),
            vmem_limit_bytes=120 * 1024 * 1024,
        ),
    )(acc, gamma.reshape(1, C), beta.reshape(1, C), y2, m2)

    return out.reshape(B, T, C)


# speedup vs baseline: 1.0321x; 1.0321x over previous
import jax
import jax.numpy as jnp
from jax.experimental import pallas as pl
from jax.experimental.pallas import tpu as pltpu

_EPS = 1e-4
_CH = 512
_NBUF = 6


def _reduce_kernel(y_hbm, m_hbm, acc_ref, ybuf, mbuf, ysem, msem):
    rows, C = y_hbm.shape
    nch = rows // _CH

    def start(c, slot):
        pltpu.make_async_copy(
            y_hbm.at[pl.ds(c * _CH, _CH), :], ybuf.at[slot], ysem.at[slot]
        ).start()
        pltpu.make_async_copy(
            m_hbm.at[pl.ds(c * _CH, _CH), :], mbuf.at[slot], msem.at[slot]
        ).start()

    for slot in range(_NBUF):
        start(slot, slot)

    dn = (((0,), (0,)), ((), ()))

    def body(c, carry):
        s, sq, n = carry
        slot = jax.lax.rem(c, _NBUF)
        pltpu.make_async_copy(
            y_hbm.at[pl.ds(c * _CH, _CH), :], ybuf.at[slot], ysem.at[slot]
        ).wait()
        pltpu.make_async_copy(
            m_hbm.at[pl.ds(c * _CH, _CH), :], mbuf.at[slot], msem.at[slot]
        ).wait()
        yb = ybuf[slot]
        w = (mbuf[slot] > 0).astype(jnp.float32)
        s = s + jax.lax.dot_general(w, yb, dn, preferred_element_type=jnp.float32)
        sq = sq + jax.lax.dot_general(
            w, yb * yb, dn, preferred_element_type=jnp.float32
        )
        n = n + jnp.sum(w)

        @pl.when(c + _NBUF < nch)
        def _():
            start(c + _NBUF, slot)

        return (s, sq, n)

    z = jnp.zeros((1, C), jnp.float32)
    s, sq, n = jax.lax.fori_loop(0, nch, body, (z, z, jnp.float32(0.0)))
    acc_ref[0:1, :] = s
    acc_ref[1:2, :] = sq
    acc_ref[2:3, :] = jnp.full((1, C), n, jnp.float32)
    acc_ref[3:8, :] = jnp.zeros((5, C), jnp.float32)


def _apply_kernel(acc_ref, g_ref, b_ref, y_ref, m_ref, o_ref):
    s = acc_ref[0, :]
    sq = acc_ref[1, :]
    n = acc_ref[2, :]
    mean = s / n
    var = (sq - s * mean) / (n - 1.0)
    std = jnp.sqrt(var)
    scale = g_ref[0, :] / (std + _EPS)
    shift = b_ref[0, :] - mean * scale
    yb = y_ref[...]
    sel = m_ref[...] > 0
    o_ref[...] = jnp.where(sel, yb * scale + shift, yb)


def kernel(y, mask, gamma, beta):
    B, T, C = y.shape
    rows = B * T
    y2 = y.reshape(rows, C)
    m2 = mask.reshape(rows, 1)

    R = 2048
    grid = rows // R

    acc = pl.pallas_call(
        _reduce_kernel,
        in_specs=[
            pl.BlockSpec(memory_space=pl.ANY),
            pl.BlockSpec(memory_space=pl.ANY),
        ],
        out_specs=pl.BlockSpec(memory_space=pltpu.VMEM),
        out_shape=jax.ShapeDtypeStruct((8, C), jnp.float32),
        scratch_shapes=[
            pltpu.VMEM((_NBUF, _CH, C), jnp.float32),
            pltpu.VMEM((_NBUF, _CH, 1), jnp.int32),
            pltpu.SemaphoreType.DMA((_NBUF,)),
            pltpu.SemaphoreType.DMA((_NBUF,)),
        ],
        compiler_params=pltpu.CompilerParams(
            vmem_limit_bytes=120 * 1024 * 1024,
        ),
    )(y2, m2)

    out = pl.pallas_call(
        _apply_kernel,
        grid=(grid,),
        in_specs=[
            pl.BlockSpec((8, C), lambda i: (0, 0)),
            pl.BlockSpec((1, C), lambda i: (0, 0)),
            pl.BlockSpec((1, C), lambda i: (0, 0)),
            pl.BlockSpec((R, C), lambda i: (i, 0)),
            pl.BlockSpec((R, 1), lambda i: (i, 0)),
        ],
        out_specs=pl.BlockSpec((R, C), lambda i: (i, 0)),
        out_shape=jax.ShapeDtypeStruct((rows, C), jnp.float32),
        compiler_params=pltpu.CompilerParams(
            dimension_semantics=("parallel",),
            vmem_limit_bytes=120 * 1024 * 1024,
        ),
    )(acc, gamma.reshape(1, C), beta.reshape(1, C), y2, m2)

    return out.reshape(B, T, C)


# speedup vs baseline: 1.0401x; 1.0077x over previous
import jax
import jax.numpy as jnp
from jax.experimental import pallas as pl
from jax.experimental.pallas import tpu as pltpu

_EPS = 1e-4
_CH = 512
_NBUF = 6
_ACH = 512
_ANBUF = 6


def _reduce_kernel(y_hbm, m_hbm, acc_ref, ybuf, mbuf, ysem, msem):
    rows, C = y_hbm.shape
    nch = rows // _CH

    def start(c, slot):
        pltpu.make_async_copy(
            y_hbm.at[pl.ds(c * _CH, _CH), :], ybuf.at[slot], ysem.at[slot]
        ).start()
        pltpu.make_async_copy(
            m_hbm.at[pl.ds(c * _CH, _CH), :], mbuf.at[slot], msem.at[slot]
        ).start()

    for slot in range(_NBUF):
        start(slot, slot)

    dn = (((0,), (0,)), ((), ()))

    def body(c, carry):
        s, sq, n = carry
        slot = jax.lax.rem(c, _NBUF)
        pltpu.make_async_copy(
            y_hbm.at[pl.ds(c * _CH, _CH), :], ybuf.at[slot], ysem.at[slot]
        ).wait()
        pltpu.make_async_copy(
            m_hbm.at[pl.ds(c * _CH, _CH), :], mbuf.at[slot], msem.at[slot]
        ).wait()
        yb = ybuf[slot]
        w = (mbuf[slot] > 0).astype(jnp.float32)
        s = s + jax.lax.dot_general(w, yb, dn, preferred_element_type=jnp.float32)
        sq = sq + jax.lax.dot_general(
            w, yb * yb, dn, preferred_element_type=jnp.float32
        )
        n = n + jnp.sum(w)

        @pl.when(c + _NBUF < nch)
        def _():
            start(c + _NBUF, slot)

        return (s, sq, n)

    z = jnp.zeros((1, C), jnp.float32)
    s, sq, n = jax.lax.fori_loop(0, nch, body, (z, z, jnp.float32(0.0)))
    acc_ref[0:1, :] = s
    acc_ref[1:2, :] = sq
    acc_ref[2:3, :] = jnp.full((1, C), n, jnp.float32)
    acc_ref[3:8, :] = jnp.zeros((5, C), jnp.float32)


def _apply_kernel(acc_ref, g_ref, b_ref, y_hbm, m_hbm, o_hbm,
                  ybuf, mbuf, obuf, ysem, msem, osem):
    rows, C = y_hbm.shape
    nch = rows // _ACH
    s = acc_ref[0, :]
    sq = acc_ref[1, :]
    n = acc_ref[2, :]
    mean = s / n
    var = (sq - s * mean) / (n - 1.0)
    std = jnp.sqrt(var)
    scale = g_ref[0, :] / (std + _EPS)
    shift = b_ref[0, :] - mean * scale

    def start_read(c, slot):
        pltpu.make_async_copy(
            y_hbm.at[pl.ds(c * _ACH, _ACH), :], ybuf.at[slot], ysem.at[slot]
        ).start()
        pltpu.make_async_copy(
            m_hbm.at[pl.ds(c * _ACH, _ACH), :], mbuf.at[slot], msem.at[slot]
        ).start()

    for slot in range(_ANBUF):
        start_read(slot, slot)

    def write_copy(c, slot):
        return pltpu.make_async_copy(
            obuf.at[slot], o_hbm.at[pl.ds(c * _ACH, _ACH), :], osem.at[slot]
        )

    def body(c, _):
        slot = jax.lax.rem(c, _ANBUF)
        pltpu.make_async_copy(
            y_hbm.at[pl.ds(c * _ACH, _ACH), :], ybuf.at[slot], ysem.at[slot]
        ).wait()
        pltpu.make_async_copy(
            m_hbm.at[pl.ds(c * _ACH, _ACH), :], mbuf.at[slot], msem.at[slot]
        ).wait()

        @pl.when(c >= _ANBUF)
        def _():
            write_copy(c - _ANBUF, slot).wait()

        yb = ybuf[slot]
        sel = mbuf[slot] > 0
        obuf[slot] = jnp.where(sel, yb * scale + shift, yb)
        write_copy(c, slot).start()

        @pl.when(c + _ANBUF < nch)
        def _():
            start_read(c + _ANBUF, slot)

        return 0

    jax.lax.fori_loop(0, nch, body, 0)
    for slot in range(_ANBUF):
        c = nch - _ANBUF + slot
        pltpu.make_async_copy(
            obuf.at[jax.lax.rem(c, _ANBUF)],
            o_hbm.at[pl.ds(c * _ACH, _ACH), :],
            osem.at[jax.lax.rem(c, _ANBUF)],
        ).wait()


def kernel(y, mask, gamma, beta):
    B, T, C = y.shape
    rows = B * T
    y2 = y.reshape(rows, C)
    m2 = mask.reshape(rows, 1)

    R = 2048
    grid = rows // R

    acc = pl.pallas_call(
        _reduce_kernel,
        in_specs=[
            pl.BlockSpec(memory_space=pl.ANY),
            pl.BlockSpec(memory_space=pl.ANY),
        ],
        out_specs=pl.BlockSpec(memory_space=pltpu.VMEM),
        out_shape=jax.ShapeDtypeStruct((8, C), jnp.float32),
        scratch_shapes=[
            pltpu.VMEM((_NBUF, _CH, C), jnp.float32),
            pltpu.VMEM((_NBUF, _CH, 1), jnp.int32),
            pltpu.SemaphoreType.DMA((_NBUF,)),
            pltpu.SemaphoreType.DMA((_NBUF,)),
        ],
        compiler_params=pltpu.CompilerParams(
            vmem_limit_bytes=120 * 1024 * 1024,
        ),
    )(y2, m2)

    out = pl.pallas_call(
        _apply_kernel,
        in_specs=[
            pl.BlockSpec(memory_space=pltpu.VMEM),
            pl.BlockSpec(memory_space=pltpu.VMEM),
            pl.BlockSpec(memory_space=pltpu.VMEM),
            pl.BlockSpec(memory_space=pl.ANY),
            pl.BlockSpec(memory_space=pl.ANY),
        ],
        out_specs=pl.BlockSpec(memory_space=pl.ANY),
        out_shape=jax.ShapeDtypeStruct((rows, C), jnp.float32),
        scratch_shapes=[
            pltpu.VMEM((_ANBUF, _ACH, C), jnp.float32),
            pltpu.VMEM((_ANBUF, _ACH, 1), jnp.int32),
            pltpu.VMEM((_ANBUF, _ACH, C), jnp.float32),
            pltpu.SemaphoreType.DMA((_ANBUF,)),
            pltpu.SemaphoreType.DMA((_ANBUF,)),
            pltpu.SemaphoreType.DMA((_ANBUF,)),
        ],
        compiler_params=pltpu.CompilerParams(
            vmem_limit_bytes=120 * 1024 * 1024,
        ),
    )(acc, gamma.reshape(1, C), beta.reshape(1, C), y2, m2)

    return out.reshape(B, T, C)
